# Initial kernel scaffold; baseline (speedup 1.0000x reference)
import jax
import jax.numpy as jnp
from jax.experimental import pallas as pl


def kernel(variate_ids, embed_table):
    raise NotImplementedError("write your pallas kernel here")


# speedup vs baseline: 5.1850x; 5.1850x over previous
import functools

import jax
import jax.numpy as jnp
from jax import lax
from jax.experimental import pallas as pl
from jax.experimental.pallas import tpu as pltpu
from jax.experimental.pallas import tpu_sc as plsc

NUM_VARIATES = 100000
D_MODEL = 128
B, T = 4096, 200
B_FLAT = B * T

_info = plsc.get_sparse_core_info()
NC, NS = _info.num_cores, _info.num_subcores
NW = NC * NS
ROWS_PER_W = B_FLAT // NW
CHUNK = 128
N_CHUNKS = ROWS_PER_W // CHUNK


def _make_gather():
    mesh = plsc.VectorSubcoreMesh(core_axis_name="c", subcore_axis_name="s")

    @functools.partial(
        pl.kernel,
        mesh=mesh,
        out_type=jax.ShapeDtypeStruct((B_FLAT, D_MODEL), jnp.float32),
        scratch_types=[
            pltpu.VMEM((CHUNK,), jnp.int32),
            pltpu.VMEM((CHUNK, D_MODEL), jnp.float32),
            pltpu.SemaphoreType.DMA,
        ],
    )
    def gather_kernel(idx_hbm, table_hbm, out_hbm, idx_v, rows_v, sem):
        wid = lax.axis_index("s") * NC + lax.axis_index("c")
        base = wid * ROWS_PER_W

        @pl.loop(0, N_CHUNKS)
        def _chunk(i):
            off = base + i * CHUNK
            pltpu.sync_copy(idx_hbm.at[pl.ds(off, CHUNK)], idx_v)
            pltpu.async_copy(table_hbm.at[idx_v], rows_v, sem).wait()
            pltpu.sync_copy(rows_v, out_hbm.at[pl.ds(off, CHUNK)])

    return gather_kernel


_gather = _make_gather()


@jax.jit
def kernel(variate_ids, embed_table):
    idx_flat = variate_ids.reshape(B_FLAT).astype(jnp.int32)
    out = _gather(idx_flat, embed_table)
    return out.reshape(B, T, D_MODEL)


# speedup vs baseline: 9.0463x; 1.7447x over previous
import functools

import jax
import jax.numpy as jnp
from jax import lax
from jax.experimental import pallas as pl
from jax.experimental.pallas import tpu as pltpu
from jax.experimental.pallas import tpu_sc as plsc

NUM_VARIATES = 100000
D_MODEL = 128
B, T = 4096, 200
B_FLAT = B * T

_info = plsc.get_sparse_core_info()
NC, NS = _info.num_cores, _info.num_subcores
NW = NC * NS
ROWS_PER_W = B_FLAT // NW
CHUNK = 128
N_CHUNKS = ROWS_PER_W // CHUNK
NBUF = 2
GROUP_ROWS = NBUF * CHUNK
GROUPS = N_CHUNKS // NBUF


def _make_gather():
    mesh = plsc.VectorSubcoreMesh(core_axis_name="c", subcore_axis_name="s")

    @functools.partial(
        pl.kernel,
        mesh=mesh,
        out_type=jax.ShapeDtypeStruct((B_FLAT, D_MODEL), jnp.float32),
        scratch_types=[
            pltpu.VMEM((N_CHUNKS, CHUNK), jnp.int32),
            pltpu.VMEM((GROUP_ROWS, D_MODEL), jnp.float32),
            pltpu.VMEM((GROUP_ROWS, D_MODEL), jnp.float32),
            pltpu.SemaphoreType.DMA,
            pltpu.SemaphoreType.DMA,
            pltpu.SemaphoreType.DMA,
            pltpu.SemaphoreType.DMA,
        ],
    )
    def gather_kernel(idx_hbm, table_hbm, out_hbm, idx_v, rows_a, rows_b,
                      gsem_a, gsem_b, ssem_a, ssem_b):
        wid = lax.axis_index("s") * NC + lax.axis_index("c")
        base = wid * ROWS_PER_W

        pltpu.sync_copy(idx_hbm.at[pl.ds(wid * N_CHUNKS, N_CHUNKS)], idx_v)

        def fire_gathers(rows, gsem, g):
            for b in range(NBUF):
                pltpu.async_copy(
                    table_hbm.at[idx_v.at[g * NBUF + b]],
                    rows.at[pl.ds(b * CHUNK, CHUNK)], gsem)

        def wait_gathers(rows, gsem):
            pltpu.make_async_copy(
                table_hbm.at[pl.ds(0, GROUP_ROWS)], rows, gsem).wait()

        def start_store(rows, ssem, g):
            pltpu.async_copy(
                rows, out_hbm.at[pl.ds(base + g * GROUP_ROWS, GROUP_ROWS)],
                ssem)

        def wait_store(rows, ssem):
            pltpu.make_async_copy(
                rows, out_hbm.at[pl.ds(base, GROUP_ROWS)], ssem).wait()

        fire_gathers(rows_a, gsem_a, 0)
        fire_gathers(rows_b, gsem_b, 1)

        @pl.loop(0, GROUPS, step=2)
        def _group(g):
            wait_gathers(rows_a, gsem_a)
            start_store(rows_a, ssem_a, g)
            wait_gathers(rows_b, gsem_b)
            start_store(rows_b, ssem_b, g + 1)

            wait_store(rows_a, ssem_a)

            @pl.when(g + 2 < GROUPS)
            def _():
                fire_gathers(rows_a, gsem_a, g + 2)

            wait_store(rows_b, ssem_b)

            @pl.when(g + 3 < GROUPS)
            def _():
                fire_gathers(rows_b, gsem_b, g + 3)

    return gather_kernel


_gather = _make_gather()


@jax.jit
def kernel(variate_ids, embed_table):
    idx = variate_ids.reshape(B_FLAT // CHUNK, CHUNK).astype(jnp.int32)
    out = _gather(idx, embed_table)
    return out.reshape(B, T, D_MODEL)


# speedup vs baseline: 9.2159x; 1.0187x over previous
import functools

import jax
import jax.numpy as jnp
from jax import lax
from jax.experimental import pallas as pl
from jax.experimental.pallas import tpu as pltpu
from jax.experimental.pallas import tpu_sc as plsc

NUM_VARIATES = 100000
D_MODEL = 128
B, T = 4096, 200
B_FLAT = B * T

_info = plsc.get_sparse_core_info()
NC, NS = _info.num_cores, _info.num_subcores
NW = NC * NS
ROWS_PER_W = B_FLAT // NW
CHUNK = 128
N_CHUNKS = ROWS_PER_W // CHUNK
BUFS = 5


def _make_gather():
    mesh = plsc.VectorSubcoreMesh(core_axis_name="c", subcore_axis_name="s")

    @functools.partial(
        pl.kernel,
        mesh=mesh,
        out_type=jax.ShapeDtypeStruct((B_FLAT, D_MODEL), jnp.float32),
        scratch_types=[
            pltpu.VMEM((N_CHUNKS, CHUNK), jnp.int32),
            pltpu.VMEM((BUFS, CHUNK, D_MODEL), jnp.float32),
            pltpu.SemaphoreType.DMA((BUFS,)),
            pltpu.SemaphoreType.DMA((BUFS,)),
        ],
    )
    def gather_kernel(idx_hbm, table_hbm, out_hbm, idx_v, rows, gsem, ssem):
        wid = lax.axis_index("s") * NC + lax.axis_index("c")
        base = wid * ROWS_PER_W

        pltpu.sync_copy(idx_hbm.at[pl.ds(wid * N_CHUNKS, N_CHUNKS)], idx_v)

        def fire_gather(b, k):
            pltpu.async_copy(table_hbm.at[idx_v.at[k]], rows.at[b],
                             gsem.at[b])

        def wait_gather(b):
            pltpu.make_async_copy(table_hbm.at[pl.ds(0, CHUNK)], rows.at[b],
                                  gsem.at[b]).wait()

        def fire_store(b, k):
            pltpu.async_copy(rows.at[b],
                             out_hbm.at[pl.ds(base + k * CHUNK, CHUNK)],
                             ssem.at[b])

        def wait_store(b):
            pltpu.make_async_copy(rows.at[b], out_hbm.at[pl.ds(base, CHUNK)],
                                  ssem.at[b]).wait()

        for b in range(BUFS):
            fire_gather(b, b)

        @pl.loop(0, N_CHUNKS, step=BUFS)
        def _round(k0):
            for b in range(BUFS):
                k = k0 + b
                wait_gather(b)
                fire_store(b, k)

                @pl.when(k + BUFS < N_CHUNKS)
                def _():
                    wait_store(b)
                    fire_gather(b, k + BUFS)

        for b in range(BUFS):
            wait_store(b)

    return gather_kernel


_gather = _make_gather()


@jax.jit
def kernel(variate_ids, embed_table):
    idx = variate_ids.reshape(B_FLAT // CHUNK, CHUNK).astype(jnp.int32)
    out = _gather(idx, embed_table)
    return out.reshape(B, T, D_MODEL)
